# baseline (speedup 1.0000x reference)
import functools

import jax
import jax.numpy as jnp
from jax import lax
from jax.experimental import pallas as pl
from jax.experimental.pallas import tpu as pltpu
from jax.experimental.pallas import tpu_sc as plsc

N = 10000
E = 320000
D_IN = 128
D_HID = 256
DH = 128
NC = 2
NS = 16
NT = NC * NS

E_HIST = E // NT

EPT = E // NS
B = 25
NB = EPT // B
GB = 40
G = NB // GB
NR = 5
QTR = NB // NR
GQ = GB // NR
CH = 624
REM = N - NS * CH

_sc_mesh = plsc.VectorSubcoreMesh(
    core_axis_name="c", subcore_axis_name="s", num_cores=NC, num_subcores=NS)


@functools.partial(
    pl.kernel,
    out_type=jax.ShapeDtypeStruct((NT, 1, N), jnp.float32),
    mesh=_sc_mesh,
    compiler_params=pltpu.CompilerParams(needs_layout_passes=False),
    scratch_types=[
        pltpu.VMEM((E_HIST,), jnp.int32),
        pltpu.VMEM((N,), jnp.float32),
        pltpu.SemaphoreType.DMA,
    ],
)
def _hist_kernel(cols_hbm, hist_hbm, col_v, hist_v, sem):
    c = lax.axis_index("c")
    s = lax.axis_index("s")
    wid = c * NS + s
    pltpu.async_copy(cols_hbm.at[pl.ds(wid * E_HIST, E_HIST)], col_v, sem).wait()

    zeros = jnp.zeros((16,), jnp.float32)

    def zero_body(i, _):
        hist_v[pl.ds(i * 16, 16)] = zeros
        return 0

    lax.fori_loop(0, N // 16, zero_body, 0)

    ones = jnp.ones((16,), jnp.float32)

    def add_body(i, _):
        idx = col_v[pl.ds(i * 16, 16)]
        plsc.addupdate_scatter(hist_v, [idx], ones)
        return 0

    lax.fori_loop(0, E_HIST // 16, add_body, 0)
    pltpu.async_copy(hist_v, hist_hbm.at[wid, 0], sem).wait()


@functools.partial(
    pl.kernel,
    out_type=jax.ShapeDtypeStruct((NC, N, DH), jnp.float32),
    mesh=_sc_mesh,
    compiler_params=pltpu.CompilerParams(needs_layout_passes=False),
    scratch_types=[
        pltpu.VMEM((2, GB, B), jnp.int32),
        pltpu.VMEM((2, GB, B), jnp.int32),
        pltpu.VMEM((NR, B, DH), jnp.float32),
        pltpu.VMEM_SHARED((N, DH), jnp.float32),
        pltpu.SemaphoreType.DMA,
        pltpu.SemaphoreType.DMA,
        pltpu.SemaphoreType.DMA,
        pltpu.SemaphoreType.DMA,
        pltpu.SemaphoreType.DMA,
        pltpu.SemaphoreType.DMA,
        pltpu.SemaphoreType.DMA,
        pltpu.SemaphoreType.DMA,
        pltpu.SemaphoreType.DMA,
        pltpu.SemaphoreType.DMA,
        pltpu.SemaphoreType.DMA,
    ],
)
def _prop_kernel(hflat_hbm, rows_hbm, cols_hbm, out_hbm,
                 rows_v, cols_v, gbuf, acc,
                 g0, g1, g2, g3, g4, s0, s1, s2, s3, s4, isem):
    c = lax.axis_index("c")
    s = lax.axis_index("s")
    gsems = (g0, g1, g2, g3, g4)
    ssems = (s0, s1, s2, s3, s4)
    ebase = s * NB
    pltpu.sync_copy(hflat_hbm.at[pl.ds(c * N + s * CH, CH), :],
                    acc.at[pl.ds(s * CH, CH), :])

    @pl.when(s == NS - 1)
    def _():
        pltpu.sync_copy(hflat_hbm.at[pl.ds(c * N + NS * CH, REM), :],
                        acc.at[pl.ds(NS * CH, REM), :])
    plsc.subcore_barrier()

    def idx_rows(m, p):
        return pltpu.make_async_copy(
            rows_hbm.at[c, pl.ds(ebase + m * GB, GB), :], rows_v.at[p], isem)

    def idx_cols(m, p):
        return pltpu.make_async_copy(
            cols_hbm.at[pl.ds(ebase + m * GB, GB), :], cols_v.at[p], isem)

    def g_desc(p, off, slot):
        return pltpu.make_async_copy(
            hflat_hbm.at[rows_v.at[p, off]], gbuf.at[slot], gsems[slot])

    def s_desc(p, off, slot):
        return pltpu.make_async_copy(
            gbuf.at[slot], acc.at[cols_v.at[p, off]], ssems[slot])

    idx_rows(0, 0).start()
    idx_cols(0, 0).start()
    idx_rows(0, 0).wait()
    idx_cols(0, 0).wait()
    for k in range(NR - 1):
        g_desc(0, k, k).start()

    def quad(qj, _):
        for k in range(NR):
            j = NR * qj + k
            p = lax.rem(lax.div(j, GB), 2)
            off = lax.rem(j, GB)
            if k == 1:
                @pl.when((lax.rem(qj, GQ) == GQ - 1) & (qj <= QTR - GQ - 1))
                def _():
                    m = lax.div(qj + 1, GQ)
                    pm = lax.rem(m, 2)
                    idx_rows(m, pm).wait()
                    idx_cols(m, pm).wait()
            if k == 3:
                @pl.when((lax.rem(qj, GQ) == 0) & (qj <= QTR - 2 * GQ))
                def _():
                    mn = lax.div(qj, GQ) + 1
                    pn = lax.rem(mn, 2)
                    idx_rows(mn, pn).start()
                    idx_cols(mn, pn).start()
            g_desc(p, off, k).wait()
            s_desc(p, off, k).start(add=True)
            jm = j - 1
            pmm = lax.rem(lax.div(jm, GB), 2)
            offm = lax.rem(jm, GB)
            if k == 0:
                @pl.when(qj > 0)
                def _():
                    s_desc(pmm, offm, (k - 1) % NR).wait()
            else:
                s_desc(pmm, offm, (k - 1) % NR).wait()
            jn = j + NR - 1
            pnn = lax.rem(lax.div(jn, GB), 2)
            offn = lax.rem(jn, GB)
            if k == 0:
                g_desc(pnn, offn, (k + NR - 1) % NR).start()
            else:
                @pl.when(qj < QTR - 1)
                def _():
                    g_desc(pnn, offn, (k + NR - 1) % NR).start()
        return 0

    lax.fori_loop(0, QTR, quad, 0)
    s_desc(lax.rem(lax.div(NB - 1, GB), 2), (NB - 1) % GB, NR - 1).wait()
    plsc.subcore_barrier()
    pltpu.sync_copy(acc.at[pl.ds(s * CH, CH), :],
                    out_hbm.at[c, pl.ds(s * CH, CH), :])

    @pl.when(s == NS - 1)
    def _():
        pltpu.sync_copy(acc.at[pl.ds(NS * CH, REM), :],
                        out_hbm.at[c, pl.ds(NS * CH, REM), :])


def _mm1_body(x_ref, w_ref, hist_ref, h_ref, dis_ref):
    deg = jnp.sum(hist_ref[...], axis=1, keepdims=True) + 1.0
    dis = lax.rsqrt(deg)
    h = jnp.dot(x_ref[...], w_ref[...], preferred_element_type=jnp.float32)
    h_ref[...] = (h * dis)[None]
    dis_ref[...] = dis


def _mm2_body(acc_ref, dis_ref, b_ref, w_ref, out_ref):
    dis = dis_ref[...]
    y = jnp.maximum(acc_ref[...] * dis + b_ref[...], 0.0)
    h = jnp.dot(y, w_ref[...], preferred_element_type=jnp.float32)
    out_ref[...] = (h * dis)[None]


def _fin_body(acc_ref, dis_ref, b_ref, out_ref):
    out_ref[...] = acc_ref[...] * dis_ref[...] + b_ref[...]


RB = 1000


def _mm1(x, W1, histT):
    return pl.pallas_call(
        _mm1_body,
        grid=(N // RB, NC),
        in_specs=[
            pl.BlockSpec((RB, D_IN), lambda i, j: (i, 0)),
            pl.BlockSpec((D_IN, DH), lambda i, j: (0, j)),
            pl.BlockSpec((RB, NT), lambda i, j: (i, 0)),
        ],
        out_specs=[
            pl.BlockSpec((1, RB, DH), lambda i, j: (j, i, 0)),
            pl.BlockSpec((RB, 1), lambda i, j: (i, 0)),
        ],
        out_shape=[
            jax.ShapeDtypeStruct((NC, N, DH), jnp.float32),
            jax.ShapeDtypeStruct((N, 1), jnp.float32),
        ],
    )(x, W1, histT)


def _mm2(acc, dis, b1, W2):
    return pl.pallas_call(
        _mm2_body,
        grid=(N // RB, NC),
        in_specs=[
            pl.BlockSpec((RB, D_HID), lambda i, j: (i, 0)),
            pl.BlockSpec((RB, 1), lambda i, j: (i, 0)),
            pl.BlockSpec((1, D_HID), lambda i, j: (0, 0)),
            pl.BlockSpec((D_HID, DH), lambda i, j: (0, j)),
        ],
        out_specs=pl.BlockSpec((1, RB, DH), lambda i, j: (j, i, 0)),
        out_shape=jax.ShapeDtypeStruct((NC, N, DH), jnp.float32),
    )(acc, dis, b1, W2)


def _fin(acc, dis, b2):
    return pl.pallas_call(
        _fin_body,
        grid=(N // RB,),
        in_specs=[
            pl.BlockSpec((RB, D_HID), lambda i: (i, 0)),
            pl.BlockSpec((RB, 1), lambda i: (i, 0)),
            pl.BlockSpec((1, D_HID), lambda i: (0, 0)),
        ],
        out_specs=pl.BlockSpec((RB, D_HID), lambda i: (i, 0)),
        out_shape=jax.ShapeDtypeStruct((N, D_HID), jnp.float32),
    )(acc, dis, b2)


@jax.jit
def kernel(x, edge_index, W1, b1, W2, b2):
    rows = edge_index[0]
    cols = edge_index[1]

    hist = _hist_kernel(cols).reshape(NT, N)
    h1p, dis = _mm1(x, W1, hist.T)

    rows_b = rows.reshape(NS * NB, B)
    rows_off = jnp.stack([rows_b, rows_b + N])
    cols_b = cols.reshape(NS * NB, B)

    acc1 = _prop_kernel(h1p.reshape(NC * N, DH), rows_off, cols_b)
    acc1 = jnp.concatenate([acc1[0], acc1[1]], axis=1)

    h2p = _mm2(acc1, dis, b1.reshape(1, D_HID), W2)
    acc2 = _prop_kernel(h2p.reshape(NC * N, DH), rows_off, cols_b)
    acc2 = jnp.concatenate([acc2[0], acc2[1]], axis=1)

    out = _fin(acc2, dis, b2.reshape(1, D_HID))
    mu, logstd = jnp.split(out, 2, axis=-1)
    return (mu, logstd)


# speedup vs baseline: 1.1170x; 1.1170x over previous
import functools

import jax
import jax.numpy as jnp
from jax import lax
from jax.experimental import pallas as pl
from jax.experimental.pallas import tpu as pltpu
from jax.experimental.pallas import tpu_sc as plsc

N = 10000
E = 320000
D_IN = 128
D_HID = 256
DH = 128
NC = 2
NS = 16
NT = NC * NS

E_HIST = E // NT

EPT = E // NS
B = 50
NB = EPT // B
GB = 40
G = NB // GB
NR = 4
QTR = NB // NR
GQ = GB // NR
CH = 624
REM = N - NS * CH

_sc_mesh = plsc.VectorSubcoreMesh(
    core_axis_name="c", subcore_axis_name="s", num_cores=NC, num_subcores=NS)


@functools.partial(
    pl.kernel,
    out_type=jax.ShapeDtypeStruct((NT, 1, N), jnp.float32),
    mesh=_sc_mesh,
    compiler_params=pltpu.CompilerParams(needs_layout_passes=False),
    scratch_types=[
        pltpu.VMEM((E_HIST,), jnp.int32),
        pltpu.VMEM((N,), jnp.float32),
        pltpu.SemaphoreType.DMA,
    ],
)
def _hist_kernel(cols_hbm, hist_hbm, col_v, hist_v, sem):
    c = lax.axis_index("c")
    s = lax.axis_index("s")
    wid = c * NS + s
    pltpu.async_copy(cols_hbm.at[pl.ds(wid * E_HIST, E_HIST)], col_v, sem).wait()

    zeros = jnp.zeros((16,), jnp.float32)

    def zero_body(i, _):
        hist_v[pl.ds(i * 16, 16)] = zeros
        return 0

    lax.fori_loop(0, N // 16, zero_body, 0)

    ones = jnp.ones((16,), jnp.float32)

    def add_body(i, _):
        idx = col_v[pl.ds(i * 16, 16)]
        plsc.addupdate_scatter(hist_v, [idx], ones)
        return 0

    lax.fori_loop(0, E_HIST // 16, add_body, 0)
    pltpu.async_copy(hist_v, hist_hbm.at[wid, 0], sem).wait()


@functools.partial(
    pl.kernel,
    out_type=jax.ShapeDtypeStruct((NC, N, DH), jnp.float32),
    mesh=_sc_mesh,
    compiler_params=pltpu.CompilerParams(needs_layout_passes=False),
    scratch_types=[
        pltpu.VMEM((2, GB, B), jnp.int32),
        pltpu.VMEM((2, GB, B), jnp.int32),
        pltpu.VMEM((NR, B, DH), jnp.float32),
        pltpu.VMEM_SHARED((N, DH), jnp.float32),
        pltpu.SemaphoreType.DMA,
        pltpu.SemaphoreType.DMA,
        pltpu.SemaphoreType.DMA,
        pltpu.SemaphoreType.DMA,
        pltpu.SemaphoreType.DMA,
        pltpu.SemaphoreType.DMA,
        pltpu.SemaphoreType.DMA,
        pltpu.SemaphoreType.DMA,
        pltpu.SemaphoreType.DMA,
    ],
)
def _prop_kernel(hflat_hbm, rows_hbm, cols_hbm, out_hbm,
                 rows_v, cols_v, gbuf, acc,
                 g0, g1, g2, g3, s0, s1, s2, s3, isem):
    c = lax.axis_index("c")
    s = lax.axis_index("s")
    gsems = (g0, g1, g2, g3)
    ssems = (s0, s1, s2, s3)
    ebase = s * NB
    pltpu.sync_copy(hflat_hbm.at[pl.ds(c * N + s * CH, CH), :],
                    acc.at[pl.ds(s * CH, CH), :])

    @pl.when(s == NS - 1)
    def _():
        pltpu.sync_copy(hflat_hbm.at[pl.ds(c * N + NS * CH, REM), :],
                        acc.at[pl.ds(NS * CH, REM), :])
    plsc.subcore_barrier()

    def idx_rows(m, p):
        return pltpu.make_async_copy(
            rows_hbm.at[c, pl.ds(ebase + m * GB, GB), :], rows_v.at[p], isem)

    def idx_cols(m, p):
        return pltpu.make_async_copy(
            cols_hbm.at[pl.ds(ebase + m * GB, GB), :], cols_v.at[p], isem)

    def g_desc(p, off, slot):
        return pltpu.make_async_copy(
            hflat_hbm.at[rows_v.at[p, off]], gbuf.at[slot], gsems[slot])

    def s_desc(p, off, slot):
        return pltpu.make_async_copy(
            gbuf.at[slot], acc.at[cols_v.at[p, off]], ssems[slot])

    idx_rows(0, 0).start()
    idx_cols(0, 0).start()
    idx_rows(0, 0).wait()
    idx_cols(0, 0).wait()
    for k in range(NR - 1):
        g_desc(0, k, k).start()

    def quad(qj, _):
        for k in range(NR):
            j = NR * qj + k
            p = lax.rem(lax.div(j, GB), 2)
            off = lax.rem(j, GB)
            if k == 1:
                @pl.when((lax.rem(qj, GQ) == GQ - 1) & (qj <= QTR - GQ - 1))
                def _():
                    m = lax.div(qj + 1, GQ)
                    pm = lax.rem(m, 2)
                    idx_rows(m, pm).wait()
                    idx_cols(m, pm).wait()
            if k == 3:
                @pl.when((lax.rem(qj, GQ) == 0) & (qj <= QTR - 2 * GQ))
                def _():
                    mn = lax.div(qj, GQ) + 1
                    pn = lax.rem(mn, 2)
                    idx_rows(mn, pn).start()
                    idx_cols(mn, pn).start()
            g_desc(p, off, k).wait()
            s_desc(p, off, k).start(add=True)
            jm = j - 1
            pmm = lax.rem(lax.div(jm, GB), 2)
            offm = lax.rem(jm, GB)
            if k == 0:
                @pl.when(qj > 0)
                def _():
                    s_desc(pmm, offm, (k - 1) % NR).wait()
            else:
                s_desc(pmm, offm, (k - 1) % NR).wait()
            jn = j + NR - 1
            pnn = lax.rem(lax.div(jn, GB), 2)
            offn = lax.rem(jn, GB)
            if k == 0:
                g_desc(pnn, offn, (k + NR - 1) % NR).start()
            else:
                @pl.when(qj < QTR - 1)
                def _():
                    g_desc(pnn, offn, (k + NR - 1) % NR).start()
        return 0

    lax.fori_loop(0, QTR, quad, 0)
    s_desc(lax.rem(lax.div(NB - 1, GB), 2), (NB - 1) % GB, NR - 1).wait()
    plsc.subcore_barrier()
    pltpu.sync_copy(acc.at[pl.ds(s * CH, CH), :],
                    out_hbm.at[c, pl.ds(s * CH, CH), :])

    @pl.when(s == NS - 1)
    def _():
        pltpu.sync_copy(acc.at[pl.ds(NS * CH, REM), :],
                        out_hbm.at[c, pl.ds(NS * CH, REM), :])


def _mm1_body(x_ref, w_ref, hist_ref, h_ref, dis_ref):
    deg = jnp.sum(hist_ref[...], axis=1, keepdims=True) + 1.0
    dis = lax.rsqrt(deg)
    h = jnp.dot(x_ref[...], w_ref[...], preferred_element_type=jnp.float32)
    h_ref[...] = (h * dis)[None]
    dis_ref[...] = dis


def _mm2_body(acc_ref, dis_ref, b_ref, w_ref, out_ref):
    dis = dis_ref[...]
    y = jnp.maximum(acc_ref[...] * dis + b_ref[...], 0.0)
    h = jnp.dot(y, w_ref[...], preferred_element_type=jnp.float32)
    out_ref[...] = (h * dis)[None]


def _fin_body(acc_ref, dis_ref, b_ref, out_ref):
    out_ref[...] = acc_ref[...] * dis_ref[...] + b_ref[...]


RB = 1000


def _mm1(x, W1, histT):
    return pl.pallas_call(
        _mm1_body,
        grid=(N // RB, NC),
        in_specs=[
            pl.BlockSpec((RB, D_IN), lambda i, j: (i, 0)),
            pl.BlockSpec((D_IN, DH), lambda i, j: (0, j)),
            pl.BlockSpec((RB, NT), lambda i, j: (i, 0)),
        ],
        out_specs=[
            pl.BlockSpec((1, RB, DH), lambda i, j: (j, i, 0)),
            pl.BlockSpec((RB, 1), lambda i, j: (i, 0)),
        ],
        out_shape=[
            jax.ShapeDtypeStruct((NC, N, DH), jnp.float32),
            jax.ShapeDtypeStruct((N, 1), jnp.float32),
        ],
    )(x, W1, histT)


def _mm2(acc, dis, b1, W2):
    return pl.pallas_call(
        _mm2_body,
        grid=(N // RB, NC),
        in_specs=[
            pl.BlockSpec((RB, D_HID), lambda i, j: (i, 0)),
            pl.BlockSpec((RB, 1), lambda i, j: (i, 0)),
            pl.BlockSpec((1, D_HID), lambda i, j: (0, 0)),
            pl.BlockSpec((D_HID, DH), lambda i, j: (0, j)),
        ],
        out_specs=pl.BlockSpec((1, RB, DH), lambda i, j: (j, i, 0)),
        out_shape=jax.ShapeDtypeStruct((NC, N, DH), jnp.float32),
    )(acc, dis, b1, W2)


def _fin(acc, dis, b2):
    return pl.pallas_call(
        _fin_body,
        grid=(N // RB,),
        in_specs=[
            pl.BlockSpec((RB, D_HID), lambda i: (i, 0)),
            pl.BlockSpec((RB, 1), lambda i: (i, 0)),
            pl.BlockSpec((1, D_HID), lambda i: (0, 0)),
        ],
        out_specs=pl.BlockSpec((RB, D_HID), lambda i: (i, 0)),
        out_shape=jax.ShapeDtypeStruct((N, D_HID), jnp.float32),
    )(acc, dis, b2)


@jax.jit
def kernel(x, edge_index, W1, b1, W2, b2):
    rows = edge_index[0]
    cols = edge_index[1]

    hist = _hist_kernel(cols).reshape(NT, N)
    h1p, dis = _mm1(x, W1, hist.T)

    rows_b = rows.reshape(NS * NB, B)
    rows_off = jnp.stack([rows_b, rows_b + N])
    cols_b = cols.reshape(NS * NB, B)

    acc1 = _prop_kernel(h1p.reshape(NC * N, DH), rows_off, cols_b)
    acc1 = jnp.concatenate([acc1[0], acc1[1]], axis=1)

    h2p = _mm2(acc1, dis, b1.reshape(1, D_HID), W2)
    acc2 = _prop_kernel(h2p.reshape(NC * N, DH), rows_off, cols_b)
    acc2 = jnp.concatenate([acc2[0], acc2[1]], axis=1)

    out = _fin(acc2, dis, b2.reshape(1, D_HID))
    mu, logstd = jnp.split(out, 2, axis=-1)
    return (mu, logstd)


# speedup vs baseline: 1.1238x; 1.0062x over previous
import functools

import jax
import jax.numpy as jnp
from jax import lax
from jax.experimental import pallas as pl
from jax.experimental.pallas import tpu as pltpu
from jax.experimental.pallas import tpu_sc as plsc

N = 10000
E = 320000
D_IN = 128
D_HID = 256
DH = 128
NC = 2
NS = 16
NT = NC * NS

E_HIST = E // NT

EPT = E // NS
B = 50
NB = EPT // B
GB = 40
G = NB // GB
NR = 4
QTR = NB // NR
GQ = GB // NR
CH = 624
REM = N - NS * CH

_sc_mesh = plsc.VectorSubcoreMesh(
    core_axis_name="c", subcore_axis_name="s", num_cores=NC, num_subcores=NS)


@functools.partial(
    pl.kernel,
    out_type=jax.ShapeDtypeStruct((NT, 1, N), jnp.float32),
    mesh=_sc_mesh,
    compiler_params=pltpu.CompilerParams(needs_layout_passes=False),
    scratch_types=[
        pltpu.VMEM((E_HIST,), jnp.int32),
        pltpu.VMEM((N,), jnp.float32),
        pltpu.SemaphoreType.DMA,
    ],
)
def _hist_kernel(cols_hbm, hist_hbm, col_v, hist_v, sem):
    c = lax.axis_index("c")
    s = lax.axis_index("s")
    wid = c * NS + s
    pltpu.async_copy(cols_hbm.at[pl.ds(wid * E_HIST, E_HIST)], col_v, sem).wait()

    zeros = jnp.zeros((16,), jnp.float32)

    def zero_body(i, _):
        hist_v[pl.ds(i * 16, 16)] = zeros
        return 0

    lax.fori_loop(0, N // 16, zero_body, 0)

    ones = jnp.ones((16,), jnp.float32)

    def add_body(i, _):
        idx = col_v[pl.ds(i * 16, 16)]
        plsc.addupdate_scatter(hist_v, [idx], ones)
        return 0

    lax.fori_loop(0, E_HIST // 16, add_body, 0)
    pltpu.async_copy(hist_v, hist_hbm.at[wid, 0], sem).wait()


@functools.partial(
    pl.kernel,
    out_type=jax.ShapeDtypeStruct((NC, N, DH), jnp.float32),
    mesh=_sc_mesh,
    compiler_params=pltpu.CompilerParams(needs_layout_passes=False),
    scratch_types=[
        pltpu.VMEM((2, GB, B), jnp.int32),
        pltpu.VMEM((2, GB, B), jnp.int32),
        pltpu.VMEM((NR, B, DH), jnp.float32),
        pltpu.VMEM_SHARED((N, DH), jnp.float32),
        pltpu.SemaphoreType.DMA,
        pltpu.SemaphoreType.DMA,
        pltpu.SemaphoreType.DMA,
        pltpu.SemaphoreType.DMA,
        pltpu.SemaphoreType.DMA,
        pltpu.SemaphoreType.DMA,
        pltpu.SemaphoreType.DMA,
        pltpu.SemaphoreType.DMA,
        pltpu.SemaphoreType.DMA,
    ],
)
def _prop_kernel(hflat_hbm, rows_hbm, cols_hbm, out_hbm,
                 rows_v, cols_v, gbuf, acc,
                 g0, g1, g2, g3, s0, s1, s2, s3, isem):
    c = lax.axis_index("c")
    s = lax.axis_index("s")
    gsems = (g0, g1, g2, g3)
    ssems = (s0, s1, s2, s3)
    ebase = s * NB
    pltpu.sync_copy(hflat_hbm.at[pl.ds(c * N + s * CH, CH), :],
                    acc.at[pl.ds(s * CH, CH), :])

    @pl.when(s == NS - 1)
    def _():
        pltpu.sync_copy(hflat_hbm.at[pl.ds(c * N + NS * CH, REM), :],
                        acc.at[pl.ds(NS * CH, REM), :])

    def idx_rows(m, p):
        return pltpu.make_async_copy(
            rows_hbm.at[c, pl.ds(ebase + m * GB, GB), :], rows_v.at[p], isem)

    def idx_cols(m, p):
        return pltpu.make_async_copy(
            cols_hbm.at[pl.ds(ebase + m * GB, GB), :], cols_v.at[p], isem)

    def g_desc(p, off, slot):
        return pltpu.make_async_copy(
            hflat_hbm.at[rows_v.at[p, off]], gbuf.at[slot], gsems[slot])

    def s_desc(p, off, slot):
        return pltpu.make_async_copy(
            gbuf.at[slot], acc.at[cols_v.at[p, off]], ssems[slot])

    idx_rows(0, 0).start()
    idx_cols(0, 0).start()
    idx_rows(0, 0).wait()
    idx_cols(0, 0).wait()
    for k in range(NR - 1):
        g_desc(0, k, k).start()
    plsc.subcore_barrier()

    def quad(qj, _):
        for k in range(NR):
            j = NR * qj + k
            p = lax.rem(lax.div(j, GB), 2)
            off = lax.rem(j, GB)
            if k == 1:
                @pl.when((lax.rem(qj, GQ) == GQ - 1) & (qj <= QTR - GQ - 1))
                def _():
                    m = lax.div(qj + 1, GQ)
                    pm = lax.rem(m, 2)
                    idx_rows(m, pm).wait()
                    idx_cols(m, pm).wait()
            if k == 3:
                @pl.when((lax.rem(qj, GQ) == 0) & (qj <= QTR - 2 * GQ))
                def _():
                    mn = lax.div(qj, GQ) + 1
                    pn = lax.rem(mn, 2)
                    idx_rows(mn, pn).start()
                    idx_cols(mn, pn).start()
            g_desc(p, off, k).wait()
            s_desc(p, off, k).start(add=True)
            jm = j - 1
            pmm = lax.rem(lax.div(jm, GB), 2)
            offm = lax.rem(jm, GB)
            if k == 0:
                @pl.when(qj > 0)
                def _():
                    s_desc(pmm, offm, (k - 1) % NR).wait()
            else:
                s_desc(pmm, offm, (k - 1) % NR).wait()
            jn = j + NR - 1
            pnn = lax.rem(lax.div(jn, GB), 2)
            offn = lax.rem(jn, GB)
            if k == 0:
                g_desc(pnn, offn, (k + NR - 1) % NR).start()
            else:
                @pl.when(qj < QTR - 1)
                def _():
                    g_desc(pnn, offn, (k + NR - 1) % NR).start()
        return 0

    lax.fori_loop(0, QTR, quad, 0)
    s_desc(lax.rem(lax.div(NB - 1, GB), 2), (NB - 1) % GB, NR - 1).wait()
    plsc.subcore_barrier()
    pltpu.sync_copy(acc.at[pl.ds(s * CH, CH), :],
                    out_hbm.at[c, pl.ds(s * CH, CH), :])

    @pl.when(s == NS - 1)
    def _():
        pltpu.sync_copy(acc.at[pl.ds(NS * CH, REM), :],
                        out_hbm.at[c, pl.ds(NS * CH, REM), :])


def _mm1_body(x_ref, w_ref, hist_ref, h_ref, dis_ref):
    deg = jnp.sum(hist_ref[...], axis=1, keepdims=True) + 1.0
    dis = lax.rsqrt(deg)
    h = jnp.dot(x_ref[...], w_ref[...], preferred_element_type=jnp.float32)
    h_ref[...] = (h * dis)[None]
    dis_ref[...] = dis


def _mm2_body(acc_ref, dis_ref, b_ref, w_ref, out_ref):
    dis = dis_ref[...]
    y = jnp.maximum(acc_ref[...] * dis + b_ref[...], 0.0)
    h = jnp.dot(y, w_ref[...], preferred_element_type=jnp.float32)
    out_ref[...] = (h * dis)[None]


def _fin_body(acc_ref, dis_ref, b_ref, out_ref):
    out_ref[...] = acc_ref[...] * dis_ref[...] + b_ref[...]


RB = 1000


def _mm1(x, W1, histT):
    return pl.pallas_call(
        _mm1_body,
        grid=(N // RB, NC),
        in_specs=[
            pl.BlockSpec((RB, D_IN), lambda i, j: (i, 0)),
            pl.BlockSpec((D_IN, DH), lambda i, j: (0, j)),
            pl.BlockSpec((RB, NT), lambda i, j: (i, 0)),
        ],
        out_specs=[
            pl.BlockSpec((1, RB, DH), lambda i, j: (j, i, 0)),
            pl.BlockSpec((RB, 1), lambda i, j: (i, 0)),
        ],
        out_shape=[
            jax.ShapeDtypeStruct((NC, N, DH), jnp.float32),
            jax.ShapeDtypeStruct((N, 1), jnp.float32),
        ],
    )(x, W1, histT)


def _mm2(acc, dis, b1, W2):
    return pl.pallas_call(
        _mm2_body,
        grid=(N // RB, NC),
        in_specs=[
            pl.BlockSpec((RB, D_HID), lambda i, j: (i, 0)),
            pl.BlockSpec((RB, 1), lambda i, j: (i, 0)),
            pl.BlockSpec((1, D_HID), lambda i, j: (0, 0)),
            pl.BlockSpec((D_HID, DH), lambda i, j: (0, j)),
        ],
        out_specs=pl.BlockSpec((1, RB, DH), lambda i, j: (j, i, 0)),
        out_shape=jax.ShapeDtypeStruct((NC, N, DH), jnp.float32),
    )(acc, dis, b1, W2)


def _fin(acc, dis, b2):
    return pl.pallas_call(
        _fin_body,
        grid=(N // RB,),
        in_specs=[
            pl.BlockSpec((RB, D_HID), lambda i: (i, 0)),
            pl.BlockSpec((RB, 1), lambda i: (i, 0)),
            pl.BlockSpec((1, D_HID), lambda i: (0, 0)),
        ],
        out_specs=pl.BlockSpec((RB, D_HID), lambda i: (i, 0)),
        out_shape=jax.ShapeDtypeStruct((N, D_HID), jnp.float32),
    )(acc, dis, b2)


@jax.jit
def kernel(x, edge_index, W1, b1, W2, b2):
    rows = edge_index[0]
    cols = edge_index[1]

    hist = _hist_kernel(cols).reshape(NT, N)
    h1p, dis = _mm1(x, W1, hist.T)

    rows_b = rows.reshape(NS * NB, B)
    rows_off = jnp.stack([rows_b, rows_b + N])
    cols_b = cols.reshape(NS * NB, B)

    acc1 = _prop_kernel(h1p.reshape(NC * N, DH), rows_off, cols_b)
    acc1 = jnp.concatenate([acc1[0], acc1[1]], axis=1)

    h2p = _mm2(acc1, dis, b1.reshape(1, D_HID), W2)
    acc2 = _prop_kernel(h2p.reshape(NC * N, DH), rows_off, cols_b)
    acc2 = jnp.concatenate([acc2[0], acc2[1]], axis=1)

    out = _fin(acc2, dis, b2.reshape(1, D_HID))
    mu, logstd = jnp.split(out, 2, axis=-1)
    return (mu, logstd)
